# Initial kernel scaffold; baseline (speedup 1.0000x reference)
import jax
import jax.numpy as jnp
from jax.experimental import pallas as pl


def kernel(feat0, feat1, feat2, feat3, feat4, x, W_conv, b_conv, W_cls, b_cls, W_reg, b_reg):
    raise NotImplementedError("write your pallas kernel here")


# speedup vs baseline: 8.8633x; 8.8633x over previous
import jax
import jax.numpy as jnp
import numpy as np
from jax.experimental import pallas as pl
from jax.experimental.pallas import tpu as pltpu

_STRIDES = (4, 8, 16, 32, 64)
_NMS_PRE = 1000
_NMS_POST = 1000
_IOU_THR = 0.7

_N_CAND = 3960
_NMS_R, _NMS_C = 8, 512
_N0 = 1000


def _conv_x(x, w, b, pad):
    y = jax.lax.conv_general_dilated(
        x, w, (1, 1), [(pad, pad), (pad, pad)],
        dimension_numbers=('NCHW', 'OIHW', 'NCHW'))
    return y + b.reshape(1, -1, 1, 1)


def _anchors_for(Hf, Wf, stride):
    ratios = jnp.array([0.5, 1.0, 2.0], dtype=jnp.float32)
    scale = 8.0
    h_r = jnp.sqrt(ratios)
    w_r = 1.0 / h_r
    ws = stride * scale * w_r
    hs = stride * scale * h_r
    base = jnp.stack([-0.5 * ws, -0.5 * hs, 0.5 * ws, 0.5 * hs], axis=1)
    sx = jnp.arange(Wf, dtype=jnp.float32) * stride
    sy = jnp.arange(Hf, dtype=jnp.float32) * stride
    yy, xx = jnp.meshgrid(sy, sx, indexing='ij')
    shifts = jnp.stack([xx.ravel(), yy.ravel(), xx.ravel(), yy.ravel()], axis=1)
    return (shifts[:, None, :] + base[None, :, :]).reshape(-1, 4)


def _decode(anchors, deltas, max_h, max_w):
    px = (anchors[:, 0] + anchors[:, 2]) * 0.5
    py = (anchors[:, 1] + anchors[:, 3]) * 0.5
    pw = anchors[:, 2] - anchors[:, 0]
    ph = anchors[:, 3] - anchors[:, 1]
    dx, dy, dw, dh = deltas[:, 0], deltas[:, 1], deltas[:, 2], deltas[:, 3]
    max_ratio = float(np.abs(np.log(16.0 / 1000.0)))
    dw = jnp.clip(dw, -max_ratio, max_ratio)
    dh = jnp.clip(dh, -max_ratio, max_ratio)
    gw = pw * jnp.exp(dw)
    gh = ph * jnp.exp(dh)
    gx = px + pw * dx
    gy = py + ph * dy
    x1 = jnp.clip(gx - 0.5 * gw, 0.0, max_w)
    y1 = jnp.clip(gy - 0.5 * gh, 0.0, max_h)
    x2 = jnp.clip(gx + 0.5 * gw, 0.0, max_w)
    y2 = jnp.clip(gy + 0.5 * gh, 0.0, max_h)
    return jnp.stack([x1, y1, x2, y2], axis=1)


def _nms_body(s_ref, x1_ref, y1_ref, x2_ref, y2_ref, out_ref):
    shape = (_NMS_R, _NMS_C)
    s = s_ref[...]
    x1 = x1_ref[...]
    y1 = y1_ref[...]
    x2 = x2_ref[...]
    y2 = y2_ref[...]
    flat = (jax.lax.broadcasted_iota(jnp.int32, shape, 0) * _NMS_C
            + jax.lax.broadcasted_iota(jnp.int32, shape, 1))
    areas = (x2 - x1) * (y2 - y1)
    neg = jnp.float32(-jnp.inf)
    big = jnp.int32(2 ** 30)

    mask0 = flat < _N0
    s0 = jnp.where(mask0, s, neg)
    m0 = jnp.max(s0)
    j0 = jnp.min(jnp.where(mask0 & (s0 == m0), flat, big))

    lane4 = jax.lax.broadcasted_iota(jnp.int32, (1, 4), 1)

    def body(i, sw):
        m = jnp.max(sw)
        j = jnp.min(jnp.where(sw == m, flat, big))
        jj = jnp.where(m == neg, j0, j)
        pick = flat == jj
        xb1 = jnp.max(jnp.where(pick, x1, neg))
        yb1 = jnp.max(jnp.where(pick, y1, neg))
        xb2 = jnp.max(jnp.where(pick, x2, neg))
        yb2 = jnp.max(jnp.where(pick, y2, neg))
        ab = (xb2 - xb1) * (yb2 - yb1)
        iw = jnp.maximum(jnp.minimum(x2, xb2) - jnp.maximum(x1, xb1), 0.0)
        ih = jnp.maximum(jnp.minimum(y2, yb2) - jnp.maximum(y1, yb1), 0.0)
        inter = iw * ih
        iou = inter / (areas + ab - inter + jnp.float32(1e-9))
        sw = jnp.where((iou > jnp.float32(_IOU_THR)) | pick, neg, sw)
        row = jnp.where(lane4 == 0, xb1,
                        jnp.where(lane4 == 1, yb1,
                                  jnp.where(lane4 == 2, xb2, yb2)))
        out_ref[pl.ds(i, 1), :] = row
        return sw

    jax.lax.fori_loop(0, _NMS_POST, body, s)


def _nms_pallas(scores, boxes):
    pad = _NMS_R * _NMS_C - _N_CAND
    s = jnp.concatenate([scores, jnp.full((pad,), -jnp.inf, jnp.float32)])
    bx = jnp.concatenate([boxes, jnp.zeros((pad, 4), jnp.float32)], axis=0)
    s2 = s.reshape(_NMS_R, _NMS_C)
    x1 = bx[:, 0].reshape(_NMS_R, _NMS_C)
    y1 = bx[:, 1].reshape(_NMS_R, _NMS_C)
    x2 = bx[:, 2].reshape(_NMS_R, _NMS_C)
    y2 = bx[:, 3].reshape(_NMS_R, _NMS_C)
    return pl.pallas_call(
        _nms_body,
        out_shape=jax.ShapeDtypeStruct((_NMS_POST, 4), jnp.float32),
    )(s2, x1, y1, x2, y2)


def kernel(feat0, feat1, feat2, feat3, feat4, x, W_conv, b_conv,
           W_cls, b_cls, W_reg, b_reg):
    img_h = float(x.shape[2])
    img_w = float(x.shape[3])
    feats = (feat0, feat1, feat2, feat3, feat4)
    mlvl_scores = []
    mlvl_props = []
    for feat, stride in zip(feats, _STRIDES):
        t = jax.nn.relu(_conv_x(feat, W_conv, b_conv, 1))
        cls = _conv_x(t, W_cls, b_cls, 0)
        reg = _conv_x(t, W_reg, b_reg, 0)
        Hf, Wf = feat.shape[2], feat.shape[3]
        anchors = _anchors_for(Hf, Wf, float(stride))
        scores = jax.nn.sigmoid(cls.transpose(0, 2, 3, 1).reshape(-1))
        deltas = reg.transpose(0, 2, 3, 1).reshape(-1, 4)
        props = _decode(anchors, deltas, img_h, img_w)
        k = min(_NMS_PRE, int(scores.shape[0]))
        top_s, top_i = jax.lax.top_k(scores, k)
        mlvl_scores.append(top_s)
        mlvl_props.append(props[top_i])
    all_scores = jnp.concatenate(mlvl_scores, axis=0)
    all_props = jnp.concatenate(mlvl_props, axis=0)
    kept = _nms_pallas(all_scores, all_props)
    return kept[None]
